# baseline (speedup 1.0000x reference)
import jax
import jax.numpy as jnp
from jax import lax
from jax.experimental import pallas as pl
from jax.experimental.pallas import tpu as pltpu
from jax.experimental.pallas import tpu_sc as plsc

N = 50000
E = 800000
D = 100
DP = 128
NP = 50688
CH = 8448
NCHUNK_PER_CORE = 3
NS = 16
RT = CH // NS
EPT = E // NS
EB = 2000
NB = EPT // EB
G = 128
CCAP = 2048
NGMAX = CCAP // G
NSCALE = 7


def _layer_body(rows, cols, vals, x, out,
                acc, rbuf, cbuf, vbuf, cpack, cval,
                ccol2, crow2, gbuf2, isem, gsem, ssem):
    cid = lax.axis_index("c")
    sid = lax.axis_index("s")
    zeros16 = jnp.zeros((16,), jnp.float32)
    izeros16 = jnp.zeros((16,), jnp.int32)

    def _zc(i, c):
        cval[pl.ds(i * 16, 16)] = zeros16
        cpack[pl.ds(i * 16, 16)] = izeros16
        return c
    lax.fori_loop(0, CCAP // 16, _zc, 0)

    def _issue_idx(bb):
        base = sid * EPT + bb * EB
        pltpu.async_copy(rows.at[pl.ds(base, EB)], rbuf, isem)
        pltpu.async_copy(cols.at[pl.ds(base, EB)], cbuf, isem)
        pltpu.async_copy(vals.at[pl.ds(base, EB)], vbuf, isem)

    def _wait_idx():
        pltpu.make_async_copy(rows.at[pl.ds(0, EB)], rbuf, isem).wait()
        pltpu.make_async_copy(rows.at[pl.ds(0, EB)], rbuf, isem).wait()
        pltpu.make_async_copy(rows.at[pl.ds(0, EB)], rbuf, isem).wait()

    def _relayout(g):
        for j in range(G // 16):
            p = cpack[pl.ds(g * G + j * 16, 16)]
            ccol2[g, pl.ds(j * 16, 16)] = p & 0xFFFF
            crow2[g, pl.ds(j * 16, 16)] = jax.lax.shift_right_logical(
                p, jnp.full((16,), 16, jnp.int32))

    for lc in range(NCHUNK_PER_CORE):
        lo = (cid * NCHUNK_PER_CORE + lc) * CH

        _issue_idx(0)

        def _zr(i, c):
            for k in range(DP // 16):
                gbuf2[0, i, pl.ds(k * 16, 16)] = zeros16
            return c
        lax.fori_loop(0, G, _zr, 0)
        for z in range(RT // G):
            pltpu.sync_copy(gbuf2.at[0], acc.at[pl.ds(sid * RT + z * G, G)])
        rem = RT % G
        if rem:
            pltpu.sync_copy(gbuf2.at[0].at[pl.ds(0, rem)],
                            acc.at[pl.ds(sid * RT + (RT // G) * G, rem)])
        plsc.subcore_barrier()

        def batch_body(b, carry):
            _wait_idx()

            def comp(i, cnt):
                r = rbuf[pl.ds(i * 16, 16)]
                c = cbuf[pl.ds(i * 16, 16)]
                m = (r >= lo) & (r < lo + CH)
                cs = plsc.cumsum(m.astype(jnp.int32))
                pos = cnt + cs - 1
                packed = jax.lax.shift_left(
                    r - lo, jnp.full((16,), 16, jnp.int32)) | c
                plsc.store_scatter(cpack, [pos], packed, mask=m)
                plsc.store_scatter(cval, [pos], vbuf[pl.ds(i * 16, 16)], mask=m)
                return cnt + cs[15]

            cnt = lax.fori_loop(0, EB // 16, comp, 0, unroll=2)
            ng = (cnt + G - 1) // G

            @pl.when(b + 1 < NB)
            def _():
                _issue_idx(b + 1)

            cval[pl.ds(cnt, 16)] = zeros16

            def zpad(k, c):
                cval[pl.ds(k * 16, 16)] = zeros16
                return c
            lax.fori_loop(cnt // 16 + 1, ng * (G // 16), zpad, 0)

            @pl.when(ng > 0)
            def _():
                _relayout(0)
                pltpu.async_copy(x.at[ccol2.at[0]], gbuf2.at[0], gsem.at[0])

            def gloop(g, c):
                buf = g & 1
                pltpu.make_async_copy(x.at[pl.ds(0, G)], gbuf2.at[buf],
                                      gsem.at[buf]).wait()

                @pl.when(g + 1 < ng)
                def _():
                    _relayout(g + 1)
                    pltpu.async_copy(x.at[ccol2.at[g + 1]], gbuf2.at[1 - buf],
                                     gsem.at[1 - buf])

                def scale(j, cc):
                    cvv = cval[pl.ds(g * G + j * 16, 16)]
                    for e in range(16):
                        v = cvv[e]
                        row = j * 16 + e
                        for k in range(NSCALE):
                            gbuf2[buf, row, pl.ds(k * 16, 16)] = (
                                gbuf2[buf, row, pl.ds(k * 16, 16)] * v)
                    return cc
                lax.fori_loop(0, G // 16, scale, 0)

                pltpu.async_copy(gbuf2.at[buf], acc.at[crow2.at[g]],
                                 ssem, add=True)
                return c
            lax.fori_loop(0, ng, gloop, 0)

            def sdrain(g, c):
                pltpu.make_async_copy(x.at[pl.ds(0, G)], gbuf2.at[0],
                                      ssem).wait()
                return c
            lax.fori_loop(0, ng, sdrain, 0)
            return carry

        lax.fori_loop(0, NB, batch_body, 0)

        plsc.subcore_barrier()
        pltpu.sync_copy(acc.at[pl.ds(sid * RT, RT)],
                        out.at[pl.ds(lo + sid * RT, RT)])
        plsc.subcore_barrier()


def _spmm(rows, cols, vals, x):
    mesh = plsc.VectorSubcoreMesh(core_axis_name="c", subcore_axis_name="s")
    f = pl.kernel(
        _layer_body,
        out_type=jax.ShapeDtypeStruct((NP, DP), jnp.float32),
        mesh=mesh,
        compiler_params=pltpu.CompilerParams(needs_layout_passes=False),
        scratch_types=[
            pltpu.VMEM_SHARED((CH, DP), jnp.float32),
            pltpu.VMEM((EB,), jnp.int32),
            pltpu.VMEM((EB,), jnp.int32),
            pltpu.VMEM((EB,), jnp.float32),
            pltpu.VMEM((CCAP,), jnp.int32),
            pltpu.VMEM((CCAP,), jnp.float32),
            pltpu.VMEM((NGMAX, G), jnp.int32),
            pltpu.VMEM((NGMAX, G), jnp.int32),
            pltpu.VMEM((2, G, DP), jnp.float32),
            pltpu.SemaphoreType.DMA,
            pltpu.SemaphoreType.DMA((2,)),
            pltpu.SemaphoreType.DMA,
        ],
    )
    return f(rows, cols, vals, x)


def _sum_body(x0, x1, x2, x3, o):
    o[...] = x0[...] + x1[:, :D] + x2[:, :D] + x3[:, :D]


def _total(x0, x1, x2, x3):
    blk = 400
    return pl.pallas_call(
        _sum_body,
        out_shape=jax.ShapeDtypeStruct((N, D), jnp.float32),
        grid=(N // blk,),
        in_specs=[
            pl.BlockSpec((blk, D), lambda i: (i, 0)),
            pl.BlockSpec((blk, DP), lambda i: (i, 0)),
            pl.BlockSpec((blk, DP), lambda i: (i, 0)),
            pl.BlockSpec((blk, DP), lambda i: (i, 0)),
        ],
        out_specs=pl.BlockSpec((blk, D), lambda i: (i, 0)),
    )(x0, x1, x2, x3)


def kernel(adj_indices, adj_values, embedding):
    rows = adj_indices[0].astype(jnp.int32)
    cols = adj_indices[1].astype(jnp.int32)
    vals = adj_values.astype(jnp.float32)
    x0 = jnp.zeros((NP, DP), jnp.float32).at[:N, :D].set(embedding)
    x1 = _spmm(rows, cols, vals, x0)
    x2 = _spmm(rows, cols, vals, x1)
    x3 = _spmm(rows, cols, vals, x2)
    return _total(embedding, x1, x2, x3)


# speedup vs baseline: 1.1085x; 1.1085x over previous
import jax
import jax.numpy as jnp
from jax import lax
from jax.experimental import pallas as pl
from jax.experimental.pallas import tpu as pltpu
from jax.experimental.pallas import tpu_sc as plsc

N = 50000
E = 800000
D = 100
DP = 128
NP = 50688
CH = 8448
NCHUNK_PER_CORE = 3
NS = 16
RT = CH // NS
EPT = E // NS
EB = 2000
NB = EPT // EB
G = 128
CCAP = 2048
NGMAX = CCAP // G
NSCALE = 7


def _layer_body(rows, cols, vals, x, out,
                acc, rbuf, cbuf, vbuf, cpack, cval,
                ccol2, crow2, gbuf, isem):
    cid = lax.axis_index("c")
    sid = lax.axis_index("s")
    zeros16 = jnp.zeros((16,), jnp.float32)
    izeros16 = jnp.zeros((16,), jnp.int32)

    def _zc(i, c):
        cval[pl.ds(i * 16, 16)] = zeros16
        cpack[pl.ds(i * 16, 16)] = izeros16
        return c
    lax.fori_loop(0, CCAP // 16, _zc, 0)

    def _issue_idx(bb):
        base = sid * EPT + bb * EB
        pltpu.async_copy(rows.at[pl.ds(base, EB)], rbuf, isem)
        pltpu.async_copy(cols.at[pl.ds(base, EB)], cbuf, isem)
        pltpu.async_copy(vals.at[pl.ds(base, EB)], vbuf, isem)

    def _wait_idx():
        pltpu.make_async_copy(rows.at[pl.ds(0, EB)], rbuf, isem).wait()
        pltpu.make_async_copy(rows.at[pl.ds(0, EB)], rbuf, isem).wait()
        pltpu.make_async_copy(rows.at[pl.ds(0, EB)], rbuf, isem).wait()

    def _relayout(g):
        for j in range(G // 16):
            p = cpack[pl.ds(g * G + j * 16, 16)]
            ccol2[g, pl.ds(j * 16, 16)] = p & 0xFFFF
            crow2[g, pl.ds(j * 16, 16)] = jax.lax.shift_right_logical(
                p, jnp.full((16,), 16, jnp.int32))

    for lc in range(NCHUNK_PER_CORE):
        lo = (cid * NCHUNK_PER_CORE + lc) * CH

        _issue_idx(0)

        def _zr(i, c):
            for k in range(DP // 16):
                gbuf[i, pl.ds(k * 16, 16)] = zeros16
            return c
        lax.fori_loop(0, G, _zr, 0)
        for z in range(RT // G):
            pltpu.sync_copy(gbuf, acc.at[pl.ds(sid * RT + z * G, G)])
        rem = RT % G
        if rem:
            pltpu.sync_copy(gbuf.at[pl.ds(0, rem)],
                            acc.at[pl.ds(sid * RT + (RT // G) * G, rem)])
        plsc.subcore_barrier()

        def batch_body(b, carry):
            _wait_idx()

            def comp(i, cnt):
                r = rbuf[pl.ds(i * 16, 16)]
                c = cbuf[pl.ds(i * 16, 16)]
                m = (r >= lo) & (r < lo + CH)
                cs = plsc.cumsum(m.astype(jnp.int32))
                pos = cnt + cs - 1
                packed = jax.lax.shift_left(
                    r - lo, jnp.full((16,), 16, jnp.int32)) | c
                plsc.store_scatter(cpack, [pos], packed, mask=m)
                plsc.store_scatter(cval, [pos], vbuf[pl.ds(i * 16, 16)], mask=m)
                return cnt + cs[15]

            cnt = lax.fori_loop(0, EB // 16, comp, 0, unroll=2)
            ng = (cnt + G - 1) // G

            @pl.when(b + 1 < NB)
            def _():
                _issue_idx(b + 1)

            cval[pl.ds(cnt, 16)] = zeros16

            def zpad(k, c):
                cval[pl.ds(k * 16, 16)] = zeros16
                return c
            lax.fori_loop(cnt // 16 + 1, ng * (G // 16), zpad, 0)

            def gloop(g, c):
                _relayout(g)
                pltpu.sync_copy(x.at[ccol2.at[g]], gbuf)

                def scale(j, cc):
                    cvv = cval[pl.ds(g * G + j * 16, 16)]
                    for e in range(16):
                        v = cvv[e]
                        row = j * 16 + e
                        for k in range(NSCALE):
                            gbuf[row, pl.ds(k * 16, 16)] = (
                                gbuf[row, pl.ds(k * 16, 16)] * v)
                    return cc
                lax.fori_loop(0, G // 16, scale, 0)

                pltpu.sync_copy(gbuf, acc.at[crow2.at[g]], add=True)
                return c
            lax.fori_loop(0, ng, gloop, 0)
            return carry

        lax.fori_loop(0, NB, batch_body, 0)

        plsc.subcore_barrier()
        pltpu.sync_copy(acc.at[pl.ds(sid * RT, RT)],
                        out.at[pl.ds(lo + sid * RT, RT)])
        plsc.subcore_barrier()


def _spmm(rows, cols, vals, x):
    mesh = plsc.VectorSubcoreMesh(core_axis_name="c", subcore_axis_name="s")
    f = pl.kernel(
        _layer_body,
        out_type=jax.ShapeDtypeStruct((NP, DP), jnp.float32),
        mesh=mesh,
        compiler_params=pltpu.CompilerParams(needs_layout_passes=False),
        scratch_types=[
            pltpu.VMEM_SHARED((CH, DP), jnp.float32),
            pltpu.VMEM((EB,), jnp.int32),
            pltpu.VMEM((EB,), jnp.int32),
            pltpu.VMEM((EB,), jnp.float32),
            pltpu.VMEM((CCAP,), jnp.int32),
            pltpu.VMEM((CCAP,), jnp.float32),
            pltpu.VMEM((NGMAX, G), jnp.int32),
            pltpu.VMEM((NGMAX, G), jnp.int32),
            pltpu.VMEM((G, DP), jnp.float32),
            pltpu.SemaphoreType.DMA,
        ],
    )
    return f(rows, cols, vals, x)


def _sum_body(x0, x1, x2, x3, o):
    o[...] = x0[...] + x1[:, :D] + x2[:, :D] + x3[:, :D]


def _total(x0, x1, x2, x3):
    blk = 400
    return pl.pallas_call(
        _sum_body,
        out_shape=jax.ShapeDtypeStruct((N, D), jnp.float32),
        grid=(N // blk,),
        in_specs=[
            pl.BlockSpec((blk, D), lambda i: (i, 0)),
            pl.BlockSpec((blk, DP), lambda i: (i, 0)),
            pl.BlockSpec((blk, DP), lambda i: (i, 0)),
            pl.BlockSpec((blk, DP), lambda i: (i, 0)),
        ],
        out_specs=pl.BlockSpec((blk, D), lambda i: (i, 0)),
    )(x0, x1, x2, x3)


def kernel(adj_indices, adj_values, embedding):
    rows = adj_indices[0].astype(jnp.int32)
    cols = adj_indices[1].astype(jnp.int32)
    vals = adj_values.astype(jnp.float32)
    x0 = jnp.zeros((NP, DP), jnp.float32).at[:N, :D].set(embedding)
    x1 = _spmm(rows, cols, vals, x0)
    x2 = _spmm(rows, cols, vals, x1)
    x3 = _spmm(rows, cols, vals, x2)
    return _total(embedding, x1, x2, x3)
